# Initial kernel scaffold; baseline (speedup 1.0000x reference)
import jax
import jax.numpy as jnp
from jax.experimental import pallas as pl


def kernel(text_input, emb_table, W, b):
    raise NotImplementedError("write your pallas kernel here")


# speedup vs baseline: 2.9121x; 2.9121x over previous
import functools

import jax
import jax.numpy as jnp
from jax import lax
from jax.experimental import pallas as pl
from jax.experimental.pallas import tpu as pltpu
from jax.experimental.pallas import tpu_sc as plsc

_VOCAB = 1000000
_EMBED = 64
_IMG = 128
_B = 16384
_L = 200

_NC = 2
_NS = 16
_NW = _NC * _NS
_SPW = _B // _NW
_HALF = _SPW // 2
_IDX_ROW = 100
_RPS = _L // _IDX_ROW
_LANES = 16
_CHUNKS = _EMBED // _LANES


def _sc_pool(table, idx2):
  mesh = plsc.VectorSubcoreMesh(core_axis_name="c", subcore_axis_name="s")

  @functools.partial(
      pl.kernel,
      out_type=jax.ShapeDtypeStruct((_B, _EMBED), jnp.float32),
      mesh=mesh,
      compiler_params=pltpu.CompilerParams(use_tc_tiling_on_sc=False),
      scratch_types=[
          pltpu.VMEM((_RPS * _HALF, _IDX_ROW), jnp.int32),
          pltpu.VMEM((2, _L, _EMBED), jnp.float32),
          pltpu.VMEM((_HALF, _EMBED), jnp.float32),
          pltpu.SemaphoreType.DMA,
          pltpu.SemaphoreType.DMA,
      ],
  )
  def pool(table_hbm, idx_hbm, out_hbm, idx_v, rows_v, out_v, sem0, sem1):
    wid = lax.axis_index("s") * _NC + lax.axis_index("c")
    sems = (sem0, sem1)

    def descr(buf, s_loc, j):
      return pltpu.make_async_copy(
          table_hbm.at[idx_v.at[s_loc * _RPS + j]],
          rows_v.at[buf, pl.ds(j * _IDX_ROW, _IDX_ROW)],
          sems[buf])

    def fire(buf, s_loc):
      for j in range(_RPS):
        descr(buf, s_loc, j).start()

    def drain(buf, s_loc):
      for j in range(_RPS):
        descr(buf, s_loc, j).wait()

    def accumulate(buf):
      def body(r, accs):
        return tuple(accs[c] + rows_v[buf, r, pl.ds(c * _LANES, _LANES)]
                     for c in range(_CHUNKS))
      zero = jnp.zeros((_LANES,), jnp.float32)
      return lax.fori_loop(0, _L, body, (zero,) * _CHUNKS, unroll=4)

    for h in range(2):
      base = wid * _SPW + h * _HALF
      pltpu.sync_copy(idx_hbm.at[pl.ds(base * _RPS, _RPS * _HALF)], idx_v)
      fire(0, 0)

      def step(i, carry):
        for bpar in range(2):
          s_loc = 2 * i + bpar
          nxt = s_loc + 1

          @pl.when(nxt < _HALF)
          def _():
            fire(1 - bpar, nxt)

          drain(bpar, s_loc)
          accs = accumulate(bpar)
          for c in range(_CHUNKS):
            out_v[s_loc, pl.ds(c * _LANES, _LANES)] = accs[c]
        return carry

      lax.fori_loop(0, _HALF // 2, step, 0)
      pltpu.sync_copy(out_v, out_hbm.at[pl.ds(base, _HALF)])

  return pool(table, idx2)


def _tc_linear(sums, w, b2):
  blk = 2048

  def body(x_ref, w_ref, b_ref, o_ref):
    o_ref[...] = lax.dot_general(
        x_ref[...], w_ref[...], (((1,), (1,)), ((), ())),
        preferred_element_type=jnp.float32) * (1.0 / _VOCAB) + b_ref[...]

  return pl.pallas_call(
      body,
      grid=(_B // blk,),
      in_specs=[
          pl.BlockSpec((blk, _EMBED), lambda i: (i, 0)),
          pl.BlockSpec((_IMG, _EMBED), lambda i: (0, 0)),
          pl.BlockSpec((1, _IMG), lambda i: (0, 0)),
      ],
      out_specs=pl.BlockSpec((blk, _IMG), lambda i: (i, 0)),
      out_shape=jax.ShapeDtypeStruct((_B, _IMG), jnp.float32),
  )(sums, w, b2)


def kernel(text_input, emb_table, W, b):
  idx2 = text_input.reshape(_B * _RPS, _IDX_ROW)
  sums = _sc_pool(emb_table, idx2)
  return _tc_linear(sums, W, b.reshape(1, _IMG))
